# Initial kernel scaffold; baseline (speedup 1.0000x reference)
import jax
import jax.numpy as jnp
from jax.experimental import pallas as pl


def kernel(x, c1w, c1b, c2w, c2b, c3w, c3b, gp_ws, gp_wn, gp_b, g_ws, g_wn, g_b, gf_ws, gf_wn, gf_b):
    raise NotImplementedError("write your pallas kernel here")


# speedup vs baseline: 30.8560x; 30.8560x over previous
import functools

import jax
import jax.numpy as jnp
from jax.experimental import pallas as pl
from jax.experimental.pallas import tpu as pltpu

WS = 16
KNN = 8
H = 224
W = 224
HC = 96
NH = H // WS
NWIN = NH * NH
P = WS * WS
BH = 8
NBLK = H // BH
BIG = 1e10


def _leaky(v):
    return jnp.where(v > 0, v, 0.05 * v)


def _conv_body(xp_ref, w_ref, b_ref, o_ref, *, cin):
    i = pl.program_id(0)
    acc = jnp.zeros((BH * W, HC), jnp.float32)
    for dy in range(3):
        rows = xp_ref[pl.ds(i * BH + dy, BH)]
        for dx in range(3):
            t = rows[:, dx:dx + W, :].reshape(BH * W, cin)
            acc = acc + jax.lax.dot(t, w_ref[dy * 3 + dx],
                                    preferred_element_type=jnp.float32)
    acc = acc + b_ref[0]
    o_ref[...] = _leaky(acc).reshape(BH, W, HC)


def _conv(xp, w9, b):
    cin = xp.shape[-1]
    return pl.pallas_call(
        functools.partial(_conv_body, cin=cin),
        grid=(NBLK,),
        in_specs=[
            pl.BlockSpec((H + 2, W + 2, cin), lambda i: (0, 0, 0)),
            pl.BlockSpec((9, cin, HC), lambda i: (0, 0, 0)),
            pl.BlockSpec((1, HC), lambda i: (0, 0)),
        ],
        out_specs=pl.BlockSpec((BH, W, HC), lambda i: (i, 0, 0)),
        out_shape=jax.ShapeDtypeStruct((H, W, HC), jnp.float32),
        compiler_params=pltpu.CompilerParams(
            dimension_semantics=("parallel",)),
    )(xp, w9, b)


def _gnn_body(f_ref, gpws_ref, gpwn_ref, gpb_ref, gws_ref, gwn_ref, gb_ref,
              gfws_ref, gfwn_ref, gfb_ref, o_ref):
    jj = jax.lax.broadcasted_iota(jnp.int32, (P, P), 1)
    ii = jax.lax.broadcasted_iota(jnp.int32, (P, P), 0)
    diag = jnp.where(ii == jj, jnp.float32(BIG), jnp.float32(0.0))

    def build_mask(f):
        g = jax.lax.dot_general(f, f, (((1,), (1,)), ((), ())),
                                preferred_element_type=jnp.float32)
        sq = jnp.sum(f * f, axis=1)
        score = sq[:, None] - 2.0 * g + sq[None, :] + diag
        mask = jnp.zeros((P, P), jnp.float32)
        for _ in range(KNN):
            m = jnp.min(score, axis=1, keepdims=True)
            cand = jnp.where(score <= m, jj, P)
            sel = jnp.min(cand, axis=1, keepdims=True)
            oh = (jj == sel).astype(jnp.float32)
            mask = mask + oh
            score = score + oh * BIG
        return mask

    def glayer(f, mask, ws, wn, b):
        agg = jax.lax.dot(mask, f, preferred_element_type=jnp.float32) * 0.125
        return (jax.lax.dot(f, ws, preferred_element_type=jnp.float32)
                + jax.lax.dot(agg, wn, preferred_element_type=jnp.float32)
                + b)

    f = f_ref[0]
    mask = build_mask(f)
    f = _leaky(glayer(f, mask, gpws_ref[...], gpwn_ref[...], gpb_ref[0]))
    for i in range(8):
        if i % 3 == 0:
            mask = build_mask(f)
        f = _leaky(glayer(f, mask, gws_ref[i], gwn_ref[i], gb_ref[i, 0]))
    agg = jax.lax.dot(mask, f, preferred_element_type=jnp.float32) * 0.125
    r = (jnp.sum(f * gfws_ref[...], axis=1)
         + jnp.sum(agg * gfwn_ref[...], axis=1) + gfb_ref[0])
    o_ref[0, 0] = jax.nn.sigmoid(r)


def _gnn(fw, gp_ws, gp_wn, gp_b, g_ws, g_wn, g_b, gf_ws, gf_wn, gf_b):
    const = lambda *idx: (lambda i: tuple(0 for _ in idx))
    return pl.pallas_call(
        _gnn_body,
        grid=(NWIN,),
        in_specs=[
            pl.BlockSpec((1, P, HC), lambda i: (i, 0, 0)),
            pl.BlockSpec((HC, HC), lambda i: (0, 0)),
            pl.BlockSpec((HC, HC), lambda i: (0, 0)),
            pl.BlockSpec((1, HC), lambda i: (0, 0)),
            pl.BlockSpec((8, HC, HC), lambda i: (0, 0, 0)),
            pl.BlockSpec((8, HC, HC), lambda i: (0, 0, 0)),
            pl.BlockSpec((8, 1, HC), lambda i: (0, 0, 0)),
            pl.BlockSpec((1, HC), lambda i: (0, 0)),
            pl.BlockSpec((1, HC), lambda i: (0, 0)),
            pl.BlockSpec((1, 1), lambda i: (0, 0)),
        ],
        out_specs=pl.BlockSpec((1, 1, P), lambda i: (i, 0, 0)),
        out_shape=jax.ShapeDtypeStruct((NWIN, 1, P), jnp.float32),
        compiler_params=pltpu.CompilerParams(
            dimension_semantics=("parallel",)),
    )(fw, gp_ws, gp_wn, gp_b, g_ws, g_wn, g_b, gf_ws, gf_wn, gf_b)


def kernel(x, c1w, c1b, c2w, c2b, c3w, c3b, gp_ws, gp_wn, gp_b,
           g_ws, g_wn, g_b, gf_ws, gf_wn, gf_b):
    x2 = x[0, 0][:, :, None]
    h = _conv(jnp.pad(x2, ((1, 1), (1, 1), (0, 0))),
              c1w.transpose(2, 3, 1, 0).reshape(9, 1, HC), c1b[None])
    h = _conv(jnp.pad(h, ((1, 1), (1, 1), (0, 0))),
              c2w.transpose(2, 3, 1, 0).reshape(9, HC, HC), c2b[None])
    h = _conv(jnp.pad(h, ((1, 1), (1, 1), (0, 0))),
              c3w.transpose(2, 3, 1, 0).reshape(9, HC, HC), c3b[None])
    fw = (h.reshape(NH, WS, NH, WS, HC)
          .transpose(0, 2, 1, 3, 4).reshape(NWIN, P, HC))
    out = _gnn(fw, gp_ws, gp_wn, gp_b[None], g_ws, g_wn, g_b[:, None, :],
               gf_ws.reshape(1, HC), gf_wn.reshape(1, HC), gf_b[None])
    y = (out.reshape(NH, NH, WS, WS)
         .transpose(0, 2, 1, 3).reshape(H, W))
    return y[None, None]


# speedup vs baseline: 77.3791x; 2.5077x over previous
import functools

import jax
import jax.numpy as jnp
from jax.experimental import pallas as pl
from jax.experimental.pallas import tpu as pltpu

WS = 16
KNN = 8
H = 224
W = 224
HC = 96
NH = H // WS
NWIN = NH * NH
P = WS * WS
BH = 8
NBLK = H // BH
WB = 2
SELKEY = 1e10
DIAGKEY = 2e10


def _leaky(v):
    return jnp.where(v > 0, v, 0.05 * v)


def _conv_body(xp_ref, w_ref, b_ref, o_ref, *, cin):
    i = pl.program_id(0)
    acc = jnp.zeros((BH * W, HC), jnp.float32)
    for dy in range(3):
        rows = xp_ref[pl.ds(i * BH + dy, BH)]
        for dx in range(3):
            t = rows[:, dx:dx + W, :].reshape(BH * W, cin)
            acc = acc + jax.lax.dot(t, w_ref[dy * 3 + dx],
                                    preferred_element_type=jnp.float32)
    acc = acc + b_ref[0]
    o_ref[...] = _leaky(acc).reshape(BH, W, HC)


def _conv(xp, w9, b):
    cin = xp.shape[-1]
    return pl.pallas_call(
        functools.partial(_conv_body, cin=cin),
        grid=(NBLK,),
        in_specs=[
            pl.BlockSpec((H + 2, W + 2, cin), lambda i: (0, 0, 0)),
            pl.BlockSpec((9, cin, HC), lambda i: (0, 0, 0)),
            pl.BlockSpec((1, HC), lambda i: (0, 0)),
        ],
        out_specs=pl.BlockSpec((BH, W, HC), lambda i: (i, 0, 0)),
        out_shape=jax.ShapeDtypeStruct((H, W, HC), jnp.float32),
        compiler_params=pltpu.CompilerParams(
            dimension_semantics=("arbitrary",)),
    )(xp, w9, b)


def _gnn_body(f_ref, gpws_ref, gpwn_ref, gpb_ref, gws_ref, gwn_ref, gb_ref,
              gfws_ref, gfwn_ref, gfb_ref, o_ref):
    jj = jax.lax.broadcasted_iota(jnp.uint32, (P, P), 1)
    ii = jax.lax.broadcasted_iota(jnp.uint32, (P, P), 0)
    diagb = ii == jj
    himask = jnp.uint32(0xFFFFFF00)

    def build_mask(fb):
        g = jax.lax.dot_general(fb, fb, (((1,), (1,)), ((), ())),
                                preferred_element_type=jnp.float32)
        ff = fb.astype(jnp.float32)
        sq = jnp.sum(ff * ff, axis=1)
        d2 = sq[:, None] - 2.0 * g + sq[None, :]
        ku = jax.lax.bitcast_convert_type(jnp.maximum(d2, 0.0), jnp.uint32)
        key = jax.lax.bitcast_convert_type((ku & himask) | jj, jnp.float32)
        key = jnp.where(diagb, DIAGKEY, key)
        for _ in range(KNN):
            m = jnp.min(key, axis=1, keepdims=True)
            key = jnp.where(key == m, SELKEY, key)
        return (key == SELKEY).astype(jnp.bfloat16)

    def aggregate(fb, masks):
        aggs = [jax.lax.dot(masks[w], fb[w * P:(w + 1) * P, :],
                            preferred_element_type=jnp.float32)
                for w in range(WB)]
        return jnp.concatenate(aggs, axis=0)

    def glayer(fb, masks, ws, wn, b):
        agg = aggregate(fb, masks).astype(jnp.bfloat16)
        out = (jax.lax.dot(fb, ws, preferred_element_type=jnp.float32)
               + jax.lax.dot(agg, wn, preferred_element_type=jnp.float32)
               + b)
        return _leaky(out).astype(jnp.bfloat16)

    fb = f_ref[...].reshape(WB * P, HC)
    masks = [build_mask(fb[w * P:(w + 1) * P, :]) for w in range(WB)]
    fb = glayer(fb, masks, gpws_ref[...], gpwn_ref[...], gpb_ref[0])
    for i in range(8):
        if i % 3 == 0:
            masks = [build_mask(fb[w * P:(w + 1) * P, :]) for w in range(WB)]
        fb = glayer(fb, masks, gws_ref[i], gwn_ref[i], gb_ref[i, 0])
    agg = aggregate(fb, masks)
    r = (jnp.sum(fb.astype(jnp.float32) * gfws_ref[...], axis=1)
         + jnp.sum(agg * gfwn_ref[...], axis=1) + gfb_ref[0])
    o_ref[...] = jax.nn.sigmoid(r).reshape(WB, 1, P)


def _gnn(fw, gp_ws, gp_wn, gp_b, g_ws, g_wn, g_b, gf_ws, gf_wn, gf_b):
    return pl.pallas_call(
        _gnn_body,
        grid=(NWIN // WB,),
        in_specs=[
            pl.BlockSpec((WB, P, HC), lambda i: (i, 0, 0)),
            pl.BlockSpec((HC, HC), lambda i: (0, 0)),
            pl.BlockSpec((HC, HC), lambda i: (0, 0)),
            pl.BlockSpec((1, HC), lambda i: (0, 0)),
            pl.BlockSpec((8, HC, HC), lambda i: (0, 0, 0)),
            pl.BlockSpec((8, HC, HC), lambda i: (0, 0, 0)),
            pl.BlockSpec((8, 1, HC), lambda i: (0, 0, 0)),
            pl.BlockSpec((1, HC), lambda i: (0, 0)),
            pl.BlockSpec((1, HC), lambda i: (0, 0)),
            pl.BlockSpec((1, 1), lambda i: (0, 0)),
        ],
        out_specs=pl.BlockSpec((WB, 1, P), lambda i: (i, 0, 0)),
        out_shape=jax.ShapeDtypeStruct((NWIN, 1, P), jnp.float32),
        compiler_params=pltpu.CompilerParams(
            dimension_semantics=("arbitrary",)),
    )(fw, gp_ws, gp_wn, gp_b, g_ws, g_wn, g_b, gf_ws, gf_wn, gf_b)


def kernel(x, c1w, c1b, c2w, c2b, c3w, c3b, gp_ws, gp_wn, gp_b,
           g_ws, g_wn, g_b, gf_ws, gf_wn, gf_b):
    x2 = x[0, 0][:, :, None]
    h = _conv(jnp.pad(x2, ((1, 1), (1, 1), (0, 0))),
              c1w.transpose(2, 3, 1, 0).reshape(9, 1, HC), c1b[None])
    h = _conv(jnp.pad(h, ((1, 1), (1, 1), (0, 0))),
              c2w.transpose(2, 3, 1, 0).reshape(9, HC, HC), c2b[None])
    h = _conv(jnp.pad(h, ((1, 1), (1, 1), (0, 0))),
              c3w.transpose(2, 3, 1, 0).reshape(9, HC, HC), c3b[None])
    fw = (h.reshape(NH, WS, NH, WS, HC)
          .transpose(0, 2, 1, 3, 4).reshape(NWIN, P, HC)
          .astype(jnp.bfloat16))
    bf = jnp.bfloat16
    out = _gnn(fw,
               gp_ws.astype(bf), (gp_wn * 0.125).astype(bf), gp_b[None],
               g_ws.astype(bf), (g_wn * 0.125).astype(bf), g_b[:, None, :],
               gf_ws.reshape(1, HC), (gf_wn * 0.125).reshape(1, HC),
               gf_b[None])
    y = (out.reshape(NH, NH, WS, WS)
         .transpose(0, 2, 1, 3).reshape(H, W))
    return y[None, None]


# speedup vs baseline: 97.6603x; 1.2621x over previous
import functools

import jax
import jax.numpy as jnp
from jax.experimental import pallas as pl
from jax.experimental.pallas import tpu as pltpu

WS = 16
KNN = 8
H = 224
W = 224
HC = 96
NH = H // WS
NWIN = NH * NH
P = WS * WS
BH = 8
NBLK = H // BH
WB = 4
SELKEY = 1e10
DIAGKEY = 2e10


def _leaky(v):
    return jnp.where(v > 0, v, 0.05 * v)


def _conv_body(xp_ref, w_ref, b_ref, o_ref, *, cin):
    i = pl.program_id(0)
    acc = jnp.zeros((BH * W, HC), jnp.float32)
    for dy in range(3):
        rows = xp_ref[pl.ds(i * BH + dy, BH)]
        for dx in range(3):
            t = rows[:, dx:dx + W, :].reshape(BH * W, cin)
            acc = acc + jax.lax.dot(t, w_ref[dy * 3 + dx],
                                    preferred_element_type=jnp.float32)
    acc = acc + b_ref[0]
    o_ref[...] = _leaky(acc).astype(jnp.bfloat16).reshape(BH, W, HC)


def _conv(xp, w9, b):
    cin = xp.shape[-1]
    return pl.pallas_call(
        functools.partial(_conv_body, cin=cin),
        grid=(NBLK,),
        in_specs=[
            pl.BlockSpec((H + 2, W + 2, cin), lambda i: (0, 0, 0)),
            pl.BlockSpec((9, cin, HC), lambda i: (0, 0, 0)),
            pl.BlockSpec((1, HC), lambda i: (0, 0)),
        ],
        out_specs=pl.BlockSpec((BH, W, HC), lambda i: (i, 0, 0)),
        out_shape=jax.ShapeDtypeStruct((H, W, HC), jnp.bfloat16),
        compiler_params=pltpu.CompilerParams(
            dimension_semantics=("arbitrary",)),
    )(xp, w9, b)


def _gnn_body(f_ref, gpws_ref, gpwn_ref, gpb_ref, gws_ref, gwn_ref, gb_ref,
              gfws_ref, gfwn_ref, gfb_ref, o_ref):
    jj = jax.lax.broadcasted_iota(jnp.uint32, (P, P), 1)
    ii = jax.lax.broadcasted_iota(jnp.uint32, (P, P), 0)
    diagb = ii == jj
    himask = jnp.uint32(0xFFFFFF00)

    def build_mask(fb):
        g = jax.lax.dot_general(fb, fb, (((1,), (1,)), ((), ())),
                                preferred_element_type=jnp.float32)
        ff = fb.astype(jnp.float32)
        sq = jnp.sum(ff * ff, axis=1)
        d2 = sq[:, None] - 2.0 * g + sq[None, :]
        ku = jax.lax.bitcast_convert_type(jnp.maximum(d2, 0.0), jnp.uint32)
        key = jax.lax.bitcast_convert_type((ku & himask) | jj, jnp.float32)
        key = jnp.where(diagb, DIAGKEY, key)
        for _ in range(KNN):
            m = jnp.min(key, axis=1, keepdims=True)
            key = jnp.where(key == m, SELKEY, key)
        return (key == SELKEY).astype(jnp.bfloat16)

    def aggregate(fb, masks):
        aggs = [jax.lax.dot(masks[w], fb[w * P:(w + 1) * P, :],
                            preferred_element_type=jnp.float32)
                for w in range(WB)]
        return jnp.concatenate(aggs, axis=0)

    def glayer(fb, masks, ws, wn, b):
        agg = aggregate(fb, masks).astype(jnp.bfloat16)
        out = (jax.lax.dot(fb, ws, preferred_element_type=jnp.float32)
               + jax.lax.dot(agg, wn, preferred_element_type=jnp.float32)
               + b)
        return _leaky(out).astype(jnp.bfloat16)

    fb = f_ref[...].reshape(WB * P, HC)
    masks = [build_mask(fb[w * P:(w + 1) * P, :]) for w in range(WB)]
    fb = glayer(fb, masks, gpws_ref[...], gpwn_ref[...], gpb_ref[0])
    for i in range(8):
        if i % 3 == 0:
            masks = [build_mask(fb[w * P:(w + 1) * P, :]) for w in range(WB)]
        fb = glayer(fb, masks, gws_ref[i], gwn_ref[i], gb_ref[i, 0])
    agg = aggregate(fb, masks)
    r = (jnp.sum(fb.astype(jnp.float32) * gfws_ref[...], axis=1)
         + jnp.sum(agg * gfwn_ref[...], axis=1) + gfb_ref[0])
    o_ref[...] = jax.nn.sigmoid(r).reshape(WB, 1, P)


def _gnn(fw, gp_ws, gp_wn, gp_b, g_ws, g_wn, g_b, gf_ws, gf_wn, gf_b):
    return pl.pallas_call(
        _gnn_body,
        grid=(NWIN // WB,),
        in_specs=[
            pl.BlockSpec((WB, P, HC), lambda i: (i, 0, 0)),
            pl.BlockSpec((HC, HC), lambda i: (0, 0)),
            pl.BlockSpec((HC, HC), lambda i: (0, 0)),
            pl.BlockSpec((1, HC), lambda i: (0, 0)),
            pl.BlockSpec((8, HC, HC), lambda i: (0, 0, 0)),
            pl.BlockSpec((8, HC, HC), lambda i: (0, 0, 0)),
            pl.BlockSpec((8, 1, HC), lambda i: (0, 0, 0)),
            pl.BlockSpec((1, HC), lambda i: (0, 0)),
            pl.BlockSpec((1, HC), lambda i: (0, 0)),
            pl.BlockSpec((1, 1), lambda i: (0, 0)),
        ],
        out_specs=pl.BlockSpec((WB, 1, P), lambda i: (i, 0, 0)),
        out_shape=jax.ShapeDtypeStruct((NWIN, 1, P), jnp.float32),
        compiler_params=pltpu.CompilerParams(
            dimension_semantics=("arbitrary",)),
    )(fw, gp_ws, gp_wn, gp_b, g_ws, g_wn, g_b, gf_ws, gf_wn, gf_b)


def kernel(x, c1w, c1b, c2w, c2b, c3w, c3b, gp_ws, gp_wn, gp_b,
           g_ws, g_wn, g_b, gf_ws, gf_wn, gf_b):
    bf = jnp.bfloat16
    x2 = x[0, 0][:, :, None].astype(bf)
    h = _conv(jnp.pad(x2, ((1, 1), (1, 1), (0, 0))),
              c1w.transpose(2, 3, 1, 0).reshape(9, 1, HC).astype(bf),
              c1b[None])
    h = _conv(jnp.pad(h, ((1, 1), (1, 1), (0, 0))),
              c2w.transpose(2, 3, 1, 0).reshape(9, HC, HC).astype(bf),
              c2b[None])
    h = _conv(jnp.pad(h, ((1, 1), (1, 1), (0, 0))),
              c3w.transpose(2, 3, 1, 0).reshape(9, HC, HC).astype(bf),
              c3b[None])
    fw = (h.reshape(NH, WS, NH, WS, HC)
          .transpose(0, 2, 1, 3, 4).reshape(NWIN, P, HC))
    out = _gnn(fw,
               gp_ws.astype(bf), (gp_wn * 0.125).astype(bf), gp_b[None],
               g_ws.astype(bf), (g_wn * 0.125).astype(bf), g_b[:, None, :],
               gf_ws.reshape(1, HC), (gf_wn * 0.125).reshape(1, HC),
               gf_b[None])
    y = (out.reshape(NH, NH, WS, WS)
         .transpose(0, 2, 1, 3).reshape(H, W))
    return y[None, None]


# speedup vs baseline: 104.1363x; 1.0663x over previous
import functools

import jax
import jax.numpy as jnp
from jax.experimental import pallas as pl
from jax.experimental.pallas import tpu as pltpu

WS = 16
KNN = 8
H = 224
W = 224
HC = 96
NH = H // WS
NWIN = NH * NH
P = WS * WS
BH = 8
NBLK = H // BH
WB = 4
SELKEY = 1e10
DIAGKEY = 2e10
PAD = 128


def _leaky(v):
    return jnp.where(v > 0, v, 0.05 * v)


def _conv_body(xp_ref, w_ref, b_ref, o_ref, *, cin):
    i = pl.program_id(0)
    acc = jnp.zeros((BH * W, HC), jnp.float32)
    for dy in range(3):
        rows = xp_ref[pl.ds(i * BH + dy, BH)]
        for dx in range(3):
            t = rows[:, dx:dx + W, :].reshape(BH * W, cin)
            acc = acc + jax.lax.dot(t.astype(jnp.bfloat16),
                                    w_ref[dy * 3 + dx],
                                    preferred_element_type=jnp.float32)
    acc = acc + b_ref[0]
    o_ref[...] = _leaky(acc).reshape(BH, W, HC)


def _conv(xp, w9, b):
    cin = xp.shape[-1]
    return pl.pallas_call(
        functools.partial(_conv_body, cin=cin),
        grid=(NBLK,),
        in_specs=[
            pl.BlockSpec((H + 2, W + 2, cin), lambda i: (0, 0, 0)),
            pl.BlockSpec((9, cin, HC), lambda i: (0, 0, 0)),
            pl.BlockSpec((1, HC), lambda i: (0, 0)),
        ],
        out_specs=pl.BlockSpec((BH, W, HC), lambda i: (i, 0, 0)),
        out_shape=jax.ShapeDtypeStruct((H, W, HC), jnp.float32),
        compiler_params=pltpu.CompilerParams(
            dimension_semantics=("arbitrary",)),
    )(xp, w9, b)


def _gnn_body(f_ref, gpws_ref, gpb_ref, gws_ref, gb_ref,
              gfws_ref, gfwn_ref, gfb_ref, o_ref):
    jj = jax.lax.broadcasted_iota(jnp.uint32, (P, P), 1)
    ii = jax.lax.broadcasted_iota(jnp.uint32, (P, P), 0)
    diagb = ii == jj
    himask = jnp.uint32(0xFFFFFF00)

    def build_mask(fb):
        g = jax.lax.dot_general(fb, fb, (((1,), (1,)), ((), ())),
                                preferred_element_type=jnp.float32)
        ff = fb.astype(jnp.float32)
        sq = jnp.sum(ff * ff, axis=1)
        d2 = sq[:, None] - 2.0 * g + sq[None, :]
        ku = jax.lax.bitcast_convert_type(d2, jnp.uint32)
        key = jax.lax.bitcast_convert_type((ku & himask) | jj, jnp.float32)
        key = jnp.where(diagb, DIAGKEY, key)
        kf = jnp.minimum(key[:, :P // 2], key[:, P // 2:])
        km = jnp.maximum(key[:, :P // 2], key[:, P // 2:])
        m = None
        for _ in range(KNN):
            m = jnp.min(kf, axis=1, keepdims=True)
            sel = kf == m
            kf = jnp.where(sel, km, kf)
            km = jnp.where(sel, SELKEY, km)
        return (key <= m).astype(jnp.bfloat16)

    def aggregate(fb, masks):
        aggs = [jax.lax.dot(masks[w], fb[w * P:(w + 1) * P, :],
                            preferred_element_type=jnp.float32)
                for w in range(WB)]
        return jnp.concatenate(aggs, axis=0)

    def glayer(fb, masks, wc, b):
        fA = jax.lax.dot(fb, wc, preferred_element_type=jnp.float32)
        fn = fA[:, PAD:].astype(jnp.bfloat16)
        aggs = [jax.lax.dot(masks[w], fn[w * P:(w + 1) * P, :],
                            preferred_element_type=jnp.float32)
                for w in range(WB)]
        agg = jnp.concatenate(aggs, axis=0)
        out = fA[:, :HC] + agg[:, :HC] + b
        return _leaky(out).astype(jnp.bfloat16)

    fb = f_ref[...].reshape(WB * P, HC)
    masks = [build_mask(fb[w * P:(w + 1) * P, :]) for w in range(WB)]
    fb = glayer(fb, masks, gpws_ref[...], gpb_ref[0])
    for i in range(8):
        if i % 3 == 0:
            masks = [build_mask(fb[w * P:(w + 1) * P, :]) for w in range(WB)]
        fb = glayer(fb, masks, gws_ref[i], gb_ref[i, 0])
    agg = aggregate(fb, masks)
    r = (jnp.sum(fb.astype(jnp.float32) * gfws_ref[...], axis=1)
         + jnp.sum(agg * gfwn_ref[...], axis=1) + gfb_ref[0])
    o_ref[...] = jax.nn.sigmoid(r).reshape(WB, 1, P)


def _gnn(fw, gp_wc, gp_b, g_wc, g_b, gf_ws, gf_wn, gf_b):
    return pl.pallas_call(
        _gnn_body,
        grid=(NWIN // WB,),
        in_specs=[
            pl.BlockSpec((WB, P, HC), lambda i: (i, 0, 0)),
            pl.BlockSpec((HC, 2 * PAD), lambda i: (0, 0)),
            pl.BlockSpec((1, HC), lambda i: (0, 0)),
            pl.BlockSpec((8, HC, 2 * PAD), lambda i: (0, 0, 0)),
            pl.BlockSpec((8, 1, HC), lambda i: (0, 0, 0)),
            pl.BlockSpec((1, HC), lambda i: (0, 0)),
            pl.BlockSpec((1, HC), lambda i: (0, 0)),
            pl.BlockSpec((1, 1), lambda i: (0, 0)),
        ],
        out_specs=pl.BlockSpec((WB, 1, P), lambda i: (i, 0, 0)),
        out_shape=jax.ShapeDtypeStruct((NWIN, 1, P), jnp.float32),
        compiler_params=pltpu.CompilerParams(
            dimension_semantics=("arbitrary",)),
    )(fw, gp_wc, gp_b, g_wc, g_b, gf_ws, gf_wn, gf_b)


def kernel(x, c1w, c1b, c2w, c2b, c3w, c3b, gp_ws, gp_wn, gp_b,
           g_ws, g_wn, g_b, gf_ws, gf_wn, gf_b):
    bf = jnp.bfloat16
    x2 = x[0, 0][:, :, None]
    h = _conv(jnp.pad(x2, ((1, 1), (1, 1), (0, 0))),
              c1w.transpose(2, 3, 1, 0).reshape(9, 1, HC).astype(bf),
              c1b[None])
    h = _conv(jnp.pad(h, ((1, 1), (1, 1), (0, 0))),
              c2w.transpose(2, 3, 1, 0).reshape(9, HC, HC).astype(bf),
              c2b[None])
    h = _conv(jnp.pad(h, ((1, 1), (1, 1), (0, 0))),
              c3w.transpose(2, 3, 1, 0).reshape(9, HC, HC).astype(bf),
              c3b[None])
    fw = (h.reshape(NH, WS, NH, WS, HC)
          .transpose(0, 2, 1, 3, 4).reshape(NWIN, P, HC)
          .astype(bf))
    gp_wc = (jnp.zeros((HC, 2 * PAD), jnp.float32)
             .at[:, :HC].set(gp_ws).at[:, PAD:PAD + HC].set(gp_wn * 0.125))
    g_wc = (jnp.zeros((8, HC, 2 * PAD), jnp.float32)
            .at[:, :, :HC].set(g_ws)
            .at[:, :, PAD:PAD + HC].set(g_wn * 0.125))
    out = _gnn(fw,
               gp_wc.astype(bf), gp_b[None],
               g_wc.astype(bf), g_b[:, None, :],
               gf_ws.reshape(1, HC), (gf_wn * 0.125).reshape(1, HC),
               gf_b[None])
    y = (out.reshape(NH, NH, WS, WS)
         .transpose(0, 2, 1, 3).reshape(H, W))
    return y[None, None]
